# baseline (speedup 1.0000x reference)
import jax
import jax.numpy as jnp
from jax.experimental import pallas as pl
from jax.experimental.pallas import tpu as pltpu
from jax.experimental.pallas import tpu_sc as plsc

_EN = 8192
_ED = 32
_M = 8192
_BETA = 0.25

_M_TILE = 4096
_N_TILE = 2048
_GM = _M // _M_TILE
_GN = _EN // _N_TILE
_HALF_TILES = _GN // 2

_GATHER_W = 128


def _tree_sumsq(x):
    s = x * x
    w = s.shape[1]
    while w > 1:
        w //= 2
        s = s[:, :w] + s[:, w:2 * w]
    return s


def _argmin_body(z_ref, et_ref, q_ref, loss_ref,
                 bA_d, bA_i, bB_d, bB_i, acc_ref):
    i = pl.program_id(0)
    j = pl.program_id(1)
    z = z_ref[...]
    et = et_ref[...]

    zsq = _tree_sumsq(z)
    es = et * et
    w = 32
    while w > 1:
        w //= 2
        es = es[:w, :] + es[w:2 * w, :]
    esq = es

    mm = jax.lax.dot_general(
        z, et, (((1,), (0,)), ((), ())),
        preferred_element_type=jnp.float32)
    dist = (zsq + esq) - 2.0 * mm

    row_min = jnp.min(dist, axis=1, keepdims=True)
    col = jax.lax.broadcasted_iota(jnp.int32, (_M_TILE, _N_TILE), 1) \
        + j * _N_TILE
    row_arg = jnp.min(
        jnp.where(dist == row_min, col, jnp.int32(2**30)),
        axis=1, keepdims=True)

    @pl.when(j == 0)
    def _():
        bA_d[...] = row_min
        bA_i[...] = row_arg

    @pl.when((j > 0) & (j < _HALF_TILES))
    def _():
        bd = bA_d[...]
        bi = bA_i[...]
        upd = row_min < bd
        bA_d[...] = jnp.where(upd, row_min, bd)
        bA_i[...] = jnp.where(upd, row_arg, bi)

    @pl.when(j == _HALF_TILES)
    def _():
        bB_d[...] = row_min
        bB_i[...] = row_arg

    @pl.when(j > _HALF_TILES)
    def _():
        bd = bB_d[...]
        bi = bB_i[...]
        upd = row_min < bd
        bB_d[...] = jnp.where(upd, row_min, bd)
        bB_i[...] = jnp.where(upd, row_arg, bi)

    @pl.when(j == _GN - 1)
    def _():
        vA = bA_d[...]
        vB = bB_d[...]
        vA_bf = vA.astype(jnp.bfloat16).astype(jnp.float32)
        win_a = vA_bf <= vB
        q_ref[...] = jnp.where(win_a, bA_i[...], bB_i[...])
        d_pick = jnp.where(win_a, vA, vB)
        part = jnp.sum(d_pick)
        tot = jnp.where(i == 0, 0.0, acc_ref[0]) + part
        acc_ref[0] = tot

        @pl.when(i == _GM - 1)
        def _():
            loss_ref[0] = tot * jnp.float32((1.0 + _BETA) / (_M * _ED))


def _tc_argmin(z_e, emb_t):
    return pl.pallas_call(
        _argmin_body,
        grid=(_GM, _GN),
        in_specs=[
            pl.BlockSpec((_M_TILE, _ED), lambda i, j: (i, 0)),
            pl.BlockSpec((_ED, _N_TILE), lambda i, j: (0, j)),
        ],
        out_specs=[
            pl.BlockSpec((_M_TILE, 1), lambda i, j: (i, 0)),
            pl.BlockSpec(memory_space=pltpu.SMEM),
        ],
        out_shape=[
            jax.ShapeDtypeStruct((_M, 1), jnp.int32),
            jax.ShapeDtypeStruct((1,), jnp.float32),
        ],
        scratch_shapes=[
            pltpu.VMEM((_M_TILE, 1), jnp.float32),
            pltpu.VMEM((_M_TILE, 1), jnp.int32),
            pltpu.VMEM((_M_TILE, 1), jnp.float32),
            pltpu.VMEM((_M_TILE, 1), jnp.int32),
            pltpu.SMEM((1,), jnp.float32),
        ],
        compiler_params=pltpu.CompilerParams(
            dimension_semantics=("arbitrary", "arbitrary")),
    )(z_e, emb_t)


def _sc_gather(embedding, idx):
    @pl.kernel(
        out_type=jax.ShapeDtypeStruct((_M, _ED), jnp.float32),
        mesh=plsc.VectorSubcoreMesh(
            core_axis_name="core", subcore_axis_name="subcore"),
        compiler_params=pltpu.CompilerParams(use_tc_tiling_on_sc=False),
    )
    def gather_kernel(e_hbm, i_hbm, o_hbm):
        def body(i_vmem, o_vmem):
            pltpu.sync_copy(e_hbm.at[i_vmem.at[0]], o_vmem)

        pltpu.emit_pipeline(
            body,
            grid=(_M // _GATHER_W,),
            in_specs=[pl.BlockSpec((1, _GATHER_W), index_map=lambda i: (0, i))],
            out_specs=[pl.BlockSpec((_GATHER_W, _ED),
                                    index_map=lambda i: (i, 0))],
            core_axis_name=("core", "subcore"),
            dimension_semantics=(pltpu.PARALLEL,),
        )(i_hbm, o_hbm)

    return gather_kernel(embedding, idx)


def kernel(z, embedding):
    z_e = z.reshape(-1, _ED)
    q, loss = _tc_argmin(z_e, embedding.T)
    z_q = _sc_gather(embedding, q.reshape(1, _M))
    return z_q.reshape(z.shape), loss.reshape(())


# speedup vs baseline: 1.1297x; 1.1297x over previous
import jax
import jax.numpy as jnp
from jax.experimental import pallas as pl
from jax.experimental.pallas import tpu as pltpu
from jax.experimental.pallas import tpu_sc as plsc

_EN = 8192
_ED = 32
_M = 8192
_BETA = 0.25

_M_TILE = 1024
_N_TILE = 4096
_GM = _M // _M_TILE
_GN = _EN // _N_TILE
_HALF_TILES = _GN // 2

_GATHER_W = 128


def _tree_sumsq(x):
    s = x * x
    w = s.shape[1]
    while w > 1:
        w //= 2
        s = s[:, :w] + s[:, w:2 * w]
    return s


def _argmin_body(z_ref, et_ref, q_ref, loss_ref,
                 bA_d, bA_i, bB_d, bB_i, acc_ref):
    i = pl.program_id(0)
    j = pl.program_id(1)
    z = z_ref[...]
    et = et_ref[...]

    zsq = _tree_sumsq(z)
    es = et * et
    w = 32
    while w > 1:
        w //= 2
        es = es[:w, :] + es[w:2 * w, :]
    esq = es

    mm = jax.lax.dot_general(
        z, et, (((1,), (0,)), ((), ())),
        preferred_element_type=jnp.float32)
    dist = (zsq + esq) - 2.0 * mm

    row_min = jnp.min(dist, axis=1, keepdims=True)
    col = jax.lax.broadcasted_iota(jnp.int32, (_M_TILE, _N_TILE), 1) \
        + j * _N_TILE
    row_arg = jnp.min(
        jnp.where(dist == row_min, col, jnp.int32(2**30)),
        axis=1, keepdims=True)

    @pl.when(j == 0)
    def _():
        bA_d[...] = row_min
        bA_i[...] = row_arg

    @pl.when((j > 0) & (j < _HALF_TILES))
    def _():
        bd = bA_d[...]
        bi = bA_i[...]
        upd = row_min < bd
        bA_d[...] = jnp.where(upd, row_min, bd)
        bA_i[...] = jnp.where(upd, row_arg, bi)

    @pl.when(j == _HALF_TILES)
    def _():
        bB_d[...] = row_min
        bB_i[...] = row_arg

    @pl.when(j > _HALF_TILES)
    def _():
        bd = bB_d[...]
        bi = bB_i[...]
        upd = row_min < bd
        bB_d[...] = jnp.where(upd, row_min, bd)
        bB_i[...] = jnp.where(upd, row_arg, bi)

    @pl.when(j == _GN - 1)
    def _():
        vA = bA_d[...]
        vB = bB_d[...]
        vA_bf = vA.astype(jnp.bfloat16).astype(jnp.float32)
        win_a = vA_bf <= vB
        q_ref[...] = jnp.where(win_a, bA_i[...], bB_i[...])
        d_pick = jnp.where(win_a, vA, vB)
        part = jnp.sum(d_pick)
        tot = jnp.where(i == 0, 0.0, acc_ref[0]) + part
        acc_ref[0] = tot

        @pl.when(i == _GM - 1)
        def _():
            loss_ref[0] = tot * jnp.float32((1.0 + _BETA) / (_M * _ED))


def _tc_argmin(z_e, emb_t):
    return pl.pallas_call(
        _argmin_body,
        grid=(_GM, _GN),
        in_specs=[
            pl.BlockSpec((_M_TILE, _ED), lambda i, j: (i, 0)),
            pl.BlockSpec((_ED, _N_TILE), lambda i, j: (0, j)),
        ],
        out_specs=[
            pl.BlockSpec((_M_TILE, 1), lambda i, j: (i, 0)),
            pl.BlockSpec(memory_space=pltpu.SMEM),
        ],
        out_shape=[
            jax.ShapeDtypeStruct((_M, 1), jnp.int32),
            jax.ShapeDtypeStruct((1,), jnp.float32),
        ],
        scratch_shapes=[
            pltpu.VMEM((_M_TILE, 1), jnp.float32),
            pltpu.VMEM((_M_TILE, 1), jnp.int32),
            pltpu.VMEM((_M_TILE, 1), jnp.float32),
            pltpu.VMEM((_M_TILE, 1), jnp.int32),
            pltpu.SMEM((1,), jnp.float32),
        ],
        compiler_params=pltpu.CompilerParams(
            dimension_semantics=("arbitrary", "arbitrary")),
    )(z_e, emb_t)


def _sc_gather(embedding, idx):
    @pl.kernel(
        out_type=jax.ShapeDtypeStruct((_M, _ED), jnp.float32),
        mesh=plsc.VectorSubcoreMesh(
            core_axis_name="core", subcore_axis_name="subcore"),
        compiler_params=pltpu.CompilerParams(use_tc_tiling_on_sc=False),
    )
    def gather_kernel(e_hbm, i_hbm, o_hbm):
        def body(i_vmem, o_vmem):
            pltpu.sync_copy(e_hbm.at[i_vmem.at[0]], o_vmem)

        pltpu.emit_pipeline(
            body,
            grid=(_M // _GATHER_W,),
            in_specs=[pl.BlockSpec((1, _GATHER_W), index_map=lambda i: (0, i))],
            out_specs=[pl.BlockSpec((_GATHER_W, _ED),
                                    index_map=lambda i: (i, 0))],
            core_axis_name=("core", "subcore"),
            dimension_semantics=(pltpu.PARALLEL,),
        )(i_hbm, o_hbm)

    return gather_kernel(embedding, idx)


def kernel(z, embedding):
    z_e = z.reshape(-1, _ED)
    q, loss = _tc_argmin(z_e, embedding.T)
    z_q = _sc_gather(embedding, q.reshape(1, _M))
    return z_q.reshape(z.shape), loss.reshape(())


# speedup vs baseline: 1.1552x; 1.0225x over previous
import jax
import jax.numpy as jnp
from jax.experimental import pallas as pl
from jax.experimental.pallas import tpu as pltpu
from jax.experimental.pallas import tpu_sc as plsc

_EN = 8192
_ED = 32
_M = 8192
_BETA = 0.25

_M_TILE = 2048
_N_TILE = 4096
_GM = _M // _M_TILE
_GN = _EN // _N_TILE
_HALF_TILES = _GN // 2

_GATHER_W = 128


def _tree_sumsq(x):
    s = x * x
    w = s.shape[1]
    while w > 1:
        w //= 2
        s = s[:, :w] + s[:, w:2 * w]
    return s


def _argmin_body(z_ref, et_ref, q_ref, loss_ref,
                 bA_d, bA_i, bB_d, bB_i, acc_ref):
    i = pl.program_id(0)
    j = pl.program_id(1)
    z = z_ref[...]
    et = et_ref[...]

    zsq = _tree_sumsq(z)
    es = et * et
    w = 32
    while w > 1:
        w //= 2
        es = es[:w, :] + es[w:2 * w, :]
    esq = es

    mm = jax.lax.dot_general(
        z, et, (((1,), (0,)), ((), ())),
        preferred_element_type=jnp.float32)
    dist = (zsq + esq) - 2.0 * mm

    row_min = jnp.min(dist, axis=1, keepdims=True)
    col = jax.lax.broadcasted_iota(jnp.int32, (_M_TILE, _N_TILE), 1) \
        + j * _N_TILE
    row_arg = jnp.min(
        jnp.where(dist == row_min, col, jnp.int32(2**30)),
        axis=1, keepdims=True)

    @pl.when(j == 0)
    def _():
        bA_d[...] = row_min
        bA_i[...] = row_arg

    @pl.when((j > 0) & (j < _HALF_TILES))
    def _():
        bd = bA_d[...]
        bi = bA_i[...]
        upd = row_min < bd
        bA_d[...] = jnp.where(upd, row_min, bd)
        bA_i[...] = jnp.where(upd, row_arg, bi)

    @pl.when(j == _HALF_TILES)
    def _():
        bB_d[...] = row_min
        bB_i[...] = row_arg

    @pl.when(j > _HALF_TILES)
    def _():
        bd = bB_d[...]
        bi = bB_i[...]
        upd = row_min < bd
        bB_d[...] = jnp.where(upd, row_min, bd)
        bB_i[...] = jnp.where(upd, row_arg, bi)

    @pl.when(j == _GN - 1)
    def _():
        vA = bA_d[...]
        vB = bB_d[...]
        vA_bf = vA.astype(jnp.bfloat16).astype(jnp.float32)
        win_a = vA_bf <= vB
        q_ref[...] = jnp.where(win_a, bA_i[...], bB_i[...])
        d_pick = jnp.where(win_a, vA, vB)
        part = jnp.sum(d_pick)
        tot = jnp.where(i == 0, 0.0, acc_ref[0]) + part
        acc_ref[0] = tot

        @pl.when(i == _GM - 1)
        def _():
            loss_ref[0] = tot * jnp.float32((1.0 + _BETA) / (_M * _ED))


def _tc_argmin(z_e, emb_t):
    return pl.pallas_call(
        _argmin_body,
        grid=(_GM, _GN),
        in_specs=[
            pl.BlockSpec((_M_TILE, _ED), lambda i, j: (i, 0)),
            pl.BlockSpec((_ED, _N_TILE), lambda i, j: (0, j)),
        ],
        out_specs=[
            pl.BlockSpec((_M_TILE, 1), lambda i, j: (i, 0)),
            pl.BlockSpec(memory_space=pltpu.SMEM),
        ],
        out_shape=[
            jax.ShapeDtypeStruct((_M, 1), jnp.int32),
            jax.ShapeDtypeStruct((1,), jnp.float32),
        ],
        scratch_shapes=[
            pltpu.VMEM((_M_TILE, 1), jnp.float32),
            pltpu.VMEM((_M_TILE, 1), jnp.int32),
            pltpu.VMEM((_M_TILE, 1), jnp.float32),
            pltpu.VMEM((_M_TILE, 1), jnp.int32),
            pltpu.SMEM((1,), jnp.float32),
        ],
        compiler_params=pltpu.CompilerParams(
            dimension_semantics=("arbitrary", "arbitrary")),
    )(z_e, emb_t)


def _sc_gather(embedding, idx):
    @pl.kernel(
        out_type=jax.ShapeDtypeStruct((_M, _ED), jnp.float32),
        mesh=plsc.VectorSubcoreMesh(
            core_axis_name="core", subcore_axis_name="subcore"),
        compiler_params=pltpu.CompilerParams(use_tc_tiling_on_sc=False),
    )
    def gather_kernel(e_hbm, i_hbm, o_hbm):
        def body(i_vmem, o_vmem):
            pltpu.sync_copy(e_hbm.at[i_vmem.at[0]], o_vmem)

        pltpu.emit_pipeline(
            body,
            grid=(_M // _GATHER_W,),
            in_specs=[pl.BlockSpec((1, _GATHER_W), index_map=lambda i: (0, i))],
            out_specs=[pl.BlockSpec((_GATHER_W, _ED),
                                    index_map=lambda i: (i, 0))],
            core_axis_name=("core", "subcore"),
            dimension_semantics=(pltpu.PARALLEL,),
        )(i_hbm, o_hbm)

    return gather_kernel(embedding, idx)


def kernel(z, embedding):
    z_e = z.reshape(-1, _ED)
    q, loss = _tc_argmin(z_e, embedding.T)
    z_q = _sc_gather(embedding, q.reshape(1, _M))
    return z_q.reshape(z.shape), loss.reshape(())
